# baseline (speedup 1.0000x reference)
import functools

import jax
import jax.numpy as jnp
from jax import lax
from jax.experimental import pallas as pl
from jax.experimental.pallas import tpu as pltpu
from jax.experimental.pallas import tpu_sc as plsc

N = 10000
E = 320000
D = 128

NC = 2
NS = 16
TILES = NC * NS

CHUNK = 128
CPT = 80
NCHUNKS = CPT * TILES
E_PAD = NCHUNKS * CHUNK
HALF = CPT // 2
N_PAD = 10240
ROWS_PT = N_PAD // NS

BN = 640


def _prep_body(wm_ref, wr_ref, wu_ref, bm_ref, br_ref, bu_ref, wcat_ref, bcat_ref):
    wu1 = wu_ref[:D, :]
    f32 = jnp.float32
    wcat_ref[:, 0 * D:1 * D] = jnp.dot(wm_ref[:D, :], wu1, preferred_element_type=f32)
    wcat_ref[:, 1 * D:2 * D] = jnp.dot(wr_ref[:D, :], wu1, preferred_element_type=f32)
    wcat_ref[:, 2 * D:3 * D] = jnp.dot(wm_ref[D:, :], wu1, preferred_element_type=f32)
    wcat_ref[:, 3 * D:4 * D] = jnp.dot(wr_ref[D:, :], wu1, preferred_element_type=f32)
    wcat_ref[:, 4 * D:5 * D] = wu_ref[D:, :]
    bcat_ref[:, 0 * D:2 * D] = jnp.zeros((1, 2 * D), f32)
    bcat_ref[:, 2 * D:3 * D] = jnp.dot(bm_ref[...], wu1, preferred_element_type=f32)
    bcat_ref[:, 3 * D:4 * D] = jnp.dot(br_ref[...], wu1, preferred_element_type=f32)
    bcat_ref[:, 4 * D:5 * D] = bu_ref[...]


_prep = pl.pallas_call(
    _prep_body,
    out_shape=(
        jax.ShapeDtypeStruct((D, 5 * D), jnp.float32),
        jax.ShapeDtypeStruct((1, 5 * D), jnp.float32),
    ),
)


def _proj_body(x_ref, w_ref, b_ref, a_ref, c_ref, bt_ref, dt_ref, u_ref):
    p = jnp.dot(x_ref[...], w_ref[...], preferred_element_type=jnp.float32) + b_ref[...]
    a_ref[...] = p[:, 0 * D:1 * D]
    c_ref[...] = p[:, 1 * D:2 * D]
    bt_ref[...] = p[:, 2 * D:3 * D]
    dt_ref[...] = p[:, 3 * D:4 * D]
    u_ref[...] = p[:, 4 * D:5 * D]


_proj = pl.pallas_call(
    _proj_body,
    grid=(N_PAD // BN,),
    in_specs=[
        pl.BlockSpec((BN, D), lambda i: (i, 0)),
        pl.BlockSpec((D, 5 * D), lambda i: (0, 0)),
        pl.BlockSpec((1, 5 * D), lambda i: (0, 0)),
    ],
    out_specs=[pl.BlockSpec((BN, D), lambda i: (i, 0)) for _ in range(5)],
    out_shape=[jax.ShapeDtypeStruct((N_PAD, D), jnp.float32) for _ in range(5)],
)


def _sc_body(a_hbm, c_hbm, fidx_hbm, tidx_hbm, z2_hbm, z1_hbm,
             s_out, deg_out,
             fblk_v, tblk_v, rows0_v, rows1_v, ones_v,
             acc_sp, indeg_sp, outdeg_sp, semf, semr, ssf, ssr):
    cid = lax.axis_index("c")
    sid = lax.axis_index("s")
    wid = cid * NS + sid
    row0 = sid * ROWS_PT

    pltpu.sync_copy(z2_hbm.at[pl.ds(row0, ROWS_PT)], acc_sp.at[pl.ds(row0, ROWS_PT)])
    pltpu.sync_copy(z1_hbm.at[pl.ds(row0, ROWS_PT)], indeg_sp.at[pl.ds(row0, ROWS_PT)])
    pltpu.sync_copy(z1_hbm.at[pl.ds(row0, ROWS_PT)], outdeg_sp.at[pl.ds(row0, ROWS_PT)])
    for j in range(CHUNK // 16):
        ones_v[pl.ds(j * 16, 16)] = jnp.ones((16,), jnp.float32)
    plsc.subcore_barrier()

    c0 = wid * CPT

    for h in range(2):
        r0 = (c0 + h * HALF) * CHUNK
        pltpu.sync_copy(fidx_hbm.at[pl.ds(r0, HALF * CHUNK)], fblk_v)
        pltpu.sync_copy(tidx_hbm.at[pl.ds(r0, HALF * CHUNK)], tblk_v)
        pltpu.async_copy(a_hbm.at[fblk_v.at[pl.ds(0, CHUNK)]], rows0_v, semf)
        pltpu.async_copy(c_hbm.at[tblk_v.at[pl.ds(0, CHUNK)]], rows1_v, semr)

        def body(g, carry):
            fcur = fblk_v.at[pl.ds(g * CHUNK, CHUNK)]
            tcur = tblk_v.at[pl.ds(g * CHUNK, CHUNK)]
            pltpu.make_async_copy(a_hbm.at[fcur], rows0_v, semf).wait()
            pltpu.async_copy(rows0_v, acc_sp.at[tcur], ssf, add=True)
            pltpu.sync_copy(ones_v, indeg_sp.at[tcur], add=True)

            pltpu.make_async_copy(c_hbm.at[tcur], rows1_v, semr).wait()
            pltpu.async_copy(rows1_v, acc_sp.at[fcur], ssr, add=True)
            pltpu.sync_copy(ones_v, outdeg_sp.at[fcur], add=True)

            @pl.when(g + 1 < HALF)
            def _next():
                fnxt = fblk_v.at[pl.ds((g + 1) * CHUNK, CHUNK)]
                tnxt = tblk_v.at[pl.ds((g + 1) * CHUNK, CHUNK)]
                pltpu.make_async_copy(rows0_v, acc_sp.at[tcur], ssf).wait()
                pltpu.async_copy(a_hbm.at[fnxt], rows0_v, semf)
                pltpu.make_async_copy(rows1_v, acc_sp.at[fcur], ssr).wait()
                pltpu.async_copy(c_hbm.at[tnxt], rows1_v, semr)

            return carry

        lax.fori_loop(0, HALF, body, 0)

        flast = fblk_v.at[pl.ds((HALF - 1) * CHUNK, CHUNK)]
        tlast = tblk_v.at[pl.ds((HALF - 1) * CHUNK, CHUNK)]
        pltpu.make_async_copy(rows0_v, acc_sp.at[tlast], ssf).wait()
        pltpu.make_async_copy(rows1_v, acc_sp.at[flast], ssr).wait()

    plsc.subcore_barrier()

    pltpu.sync_copy(acc_sp.at[pl.ds(row0, ROWS_PT)], s_out.at[cid, pl.ds(row0, ROWS_PT)])
    pltpu.sync_copy(indeg_sp.at[pl.ds(row0, ROWS_PT)], deg_out.at[cid, 0, pl.ds(row0, ROWS_PT)])
    pltpu.sync_copy(outdeg_sp.at[pl.ds(row0, ROWS_PT)], deg_out.at[cid, 1, pl.ds(row0, ROWS_PT)])


_sc_scatter = functools.partial(
    pl.kernel,
    out_type=(
        jax.ShapeDtypeStruct((NC, N_PAD, D), jnp.float32),
        jax.ShapeDtypeStruct((NC, 2, N_PAD), jnp.float32),
    ),
    mesh=plsc.VectorSubcoreMesh(core_axis_name="c", subcore_axis_name="s"),
    scratch_types=[
        pltpu.VMEM((HALF * CHUNK,), jnp.int32),
        pltpu.VMEM((HALF * CHUNK,), jnp.int32),
        pltpu.VMEM((CHUNK, D), jnp.float32),
        pltpu.VMEM((CHUNK, D), jnp.float32),
        pltpu.VMEM((CHUNK,), jnp.float32),
        pltpu.VMEM_SHARED((N_PAD, D), jnp.float32),
        pltpu.VMEM_SHARED((N_PAD,), jnp.float32),
        pltpu.VMEM_SHARED((N_PAD,), jnp.float32),
        pltpu.SemaphoreType.DMA,
        pltpu.SemaphoreType.DMA,
        pltpu.SemaphoreType.DMA,
        pltpu.SemaphoreType.DMA,
    ],
)(_sc_body)


def _comb_body(s0_ref, s1_ref, u_ref, bt_ref, dt_ref, di_ref, do_ref, o_ref):
    o_ref[...] = (s0_ref[...] + s1_ref[...] + u_ref[...]
                  + di_ref[...] * bt_ref[...] + do_ref[...] * dt_ref[...])


_combine = pl.pallas_call(
    _comb_body,
    grid=(N_PAD // BN,),
    in_specs=[pl.BlockSpec((BN, D), lambda i: (i, 0)) for _ in range(5)]
    + [pl.BlockSpec((BN, 1), lambda i: (i, 0)) for _ in range(2)],
    out_specs=pl.BlockSpec((BN, D), lambda i: (i, 0)),
    out_shape=jax.ShapeDtypeStruct((N_PAD, D), jnp.float32),
)


def kernel(node_states, from_idx, to_idx, W_msg, b_msg, W_rmsg, b_rmsg, W_upd, b_upd):
    wcat, bcat = _prep(W_msg, W_rmsg, W_upd,
                       b_msg.reshape(1, D), b_rmsg.reshape(1, D), b_upd.reshape(1, D))
    ns_pad = jnp.pad(node_states, ((0, N_PAD - N), (0, 0)))
    a, c, bt, dt, u = _proj(ns_pad, wcat, bcat)
    idx_pad = N + (jnp.arange(E_PAD - E, dtype=jnp.int32) % (N_PAD - N))
    fi = jnp.concatenate([from_idx, idx_pad])
    ti = jnp.concatenate([to_idx, idx_pad])
    z2 = jnp.zeros((N_PAD, D), jnp.float32)
    z1 = jnp.zeros((N_PAD,), jnp.float32)
    s, degs = _sc_scatter(a, c, fi, ti, z2, z1)
    indeg = (degs[0, 0] + degs[1, 0]).reshape(N_PAD, 1)
    outdeg = (degs[0, 1] + degs[1, 1]).reshape(N_PAD, 1)
    return _combine(s[0], s[1], u, bt, dt, indeg, outdeg)[:N]


# speedup vs baseline: 1.2649x; 1.2649x over previous
import functools

import jax
import jax.numpy as jnp
from jax import lax
from jax.experimental import pallas as pl
from jax.experimental.pallas import tpu as pltpu
from jax.experimental.pallas import tpu_sc as plsc

N = 10000
E = 320000
D = 128

NC = 2
NS = 16
TILES = NC * NS

CHUNK = 128
CPT = 80
NCHUNKS = CPT * TILES
E_PAD = NCHUNKS * CHUNK
HALF = CPT // 2
N_PAD = 10240
ROWS_PT = N_PAD // NS

BN = 640


def _prep_body(wm_ref, wr_ref, wu_ref, bm_ref, br_ref, bu_ref, wcat_ref, bcat_ref):
    wu1 = wu_ref[:D, :]
    f32 = jnp.float32
    wcat_ref[:, 0 * D:1 * D] = jnp.dot(wm_ref[:D, :], wu1, preferred_element_type=f32)
    wcat_ref[:, 1 * D:2 * D] = jnp.dot(wr_ref[:D, :], wu1, preferred_element_type=f32)
    wcat_ref[:, 2 * D:3 * D] = jnp.dot(wm_ref[D:, :], wu1, preferred_element_type=f32)
    wcat_ref[:, 3 * D:4 * D] = jnp.dot(wr_ref[D:, :], wu1, preferred_element_type=f32)
    wcat_ref[:, 4 * D:5 * D] = wu_ref[D:, :]
    bcat_ref[:, 0 * D:2 * D] = jnp.zeros((1, 2 * D), f32)
    bcat_ref[:, 2 * D:3 * D] = jnp.dot(bm_ref[...], wu1, preferred_element_type=f32)
    bcat_ref[:, 3 * D:4 * D] = jnp.dot(br_ref[...], wu1, preferred_element_type=f32)
    bcat_ref[:, 4 * D:5 * D] = bu_ref[...]


_prep = pl.pallas_call(
    _prep_body,
    out_shape=(
        jax.ShapeDtypeStruct((D, 5 * D), jnp.float32),
        jax.ShapeDtypeStruct((1, 5 * D), jnp.float32),
    ),
)


def _proj_body(x_ref, w_ref, b_ref, a_ref, c_ref, bt_ref, dt_ref, u_ref):
    p = jnp.dot(x_ref[...], w_ref[...], preferred_element_type=jnp.float32) + b_ref[...]
    a_ref[...] = p[:, 0 * D:1 * D]
    c_ref[...] = p[:, 1 * D:2 * D]
    bt_ref[...] = p[:, 2 * D:3 * D]
    dt_ref[...] = p[:, 3 * D:4 * D]
    u_ref[...] = p[:, 4 * D:5 * D]


_proj = pl.pallas_call(
    _proj_body,
    grid=(N_PAD // BN,),
    in_specs=[
        pl.BlockSpec((BN, D), lambda i: (i, 0)),
        pl.BlockSpec((D, 5 * D), lambda i: (0, 0)),
        pl.BlockSpec((1, 5 * D), lambda i: (0, 0)),
    ],
    out_specs=[pl.BlockSpec((BN, D), lambda i: (i, 0)) for _ in range(5)],
    out_shape=[jax.ShapeDtypeStruct((N_PAD, D), jnp.float32) for _ in range(5)],
)


def _sc_body(a_hbm, c_hbm, fidx_hbm, tidx_hbm, z2_hbm, z1_hbm,
             s_out, deg_out,
             fblk_v, tblk_v, rows0_v, rows1_v, ones_v,
             acc_sp, indeg_sp, outdeg_sp, semf, semr):
    cid = lax.axis_index("c")
    sid = lax.axis_index("s")
    wid = cid * NS + sid
    row0 = sid * ROWS_PT

    pltpu.sync_copy(z2_hbm.at[pl.ds(row0, ROWS_PT)], acc_sp.at[pl.ds(row0, ROWS_PT)])
    pltpu.sync_copy(z1_hbm.at[pl.ds(row0, ROWS_PT)], indeg_sp.at[pl.ds(row0, ROWS_PT)])
    pltpu.sync_copy(z1_hbm.at[pl.ds(row0, ROWS_PT)], outdeg_sp.at[pl.ds(row0, ROWS_PT)])
    for j in range(CHUNK // 16):
        ones_v[pl.ds(j * 16, 16)] = jnp.ones((16,), jnp.float32)
    plsc.subcore_barrier()

    c0 = wid * CPT

    for h in range(2):
        r0 = (c0 + h * HALF) * CHUNK
        pltpu.sync_copy(fidx_hbm.at[pl.ds(r0, HALF * CHUNK)], fblk_v)
        pltpu.sync_copy(tidx_hbm.at[pl.ds(r0, HALF * CHUNK)], tblk_v)
        pltpu.async_copy(a_hbm.at[fblk_v.at[pl.ds(0, CHUNK)]], rows0_v, semf)
        pltpu.async_copy(c_hbm.at[tblk_v.at[pl.ds(0, CHUNK)]], rows1_v, semr)

        def body(g, carry):
            fcur = fblk_v.at[pl.ds(g * CHUNK, CHUNK)]
            tcur = tblk_v.at[pl.ds(g * CHUNK, CHUNK)]
            pltpu.make_async_copy(a_hbm.at[fcur], rows0_v, semf).wait()
            pltpu.sync_copy(rows0_v, acc_sp.at[tcur], add=True)
            pltpu.sync_copy(ones_v, indeg_sp.at[tcur], add=True)

            @pl.when(g + 1 < HALF)
            def _nxtf():
                pltpu.async_copy(a_hbm.at[fblk_v.at[pl.ds((g + 1) * CHUNK, CHUNK)]],
                                 rows0_v, semf)

            pltpu.make_async_copy(c_hbm.at[tcur], rows1_v, semr).wait()
            pltpu.sync_copy(rows1_v, acc_sp.at[fcur], add=True)
            pltpu.sync_copy(ones_v, outdeg_sp.at[fcur], add=True)

            @pl.when(g + 1 < HALF)
            def _nxtr():
                pltpu.async_copy(c_hbm.at[tblk_v.at[pl.ds((g + 1) * CHUNK, CHUNK)]],
                                 rows1_v, semr)

            return carry

        lax.fori_loop(0, HALF, body, 0)

    plsc.subcore_barrier()

    pltpu.sync_copy(acc_sp.at[pl.ds(row0, ROWS_PT)], s_out.at[cid, pl.ds(row0, ROWS_PT)])
    pltpu.sync_copy(indeg_sp.at[pl.ds(row0, ROWS_PT)], deg_out.at[cid, 0, pl.ds(row0, ROWS_PT)])
    pltpu.sync_copy(outdeg_sp.at[pl.ds(row0, ROWS_PT)], deg_out.at[cid, 1, pl.ds(row0, ROWS_PT)])


_sc_scatter = functools.partial(
    pl.kernel,
    out_type=(
        jax.ShapeDtypeStruct((NC, N_PAD, D), jnp.float32),
        jax.ShapeDtypeStruct((NC, 2, N_PAD), jnp.float32),
    ),
    mesh=plsc.VectorSubcoreMesh(core_axis_name="c", subcore_axis_name="s"),
    scratch_types=[
        pltpu.VMEM((HALF * CHUNK,), jnp.int32),
        pltpu.VMEM((HALF * CHUNK,), jnp.int32),
        pltpu.VMEM((CHUNK, D), jnp.float32),
        pltpu.VMEM((CHUNK, D), jnp.float32),
        pltpu.VMEM((CHUNK,), jnp.float32),
        pltpu.VMEM_SHARED((N_PAD, D), jnp.float32),
        pltpu.VMEM_SHARED((N_PAD,), jnp.float32),
        pltpu.VMEM_SHARED((N_PAD,), jnp.float32),
        pltpu.SemaphoreType.DMA,
        pltpu.SemaphoreType.DMA,
    ],
)(_sc_body)


def _comb_body(s0_ref, s1_ref, u_ref, bt_ref, dt_ref, di_ref, do_ref, o_ref):
    o_ref[...] = (s0_ref[...] + s1_ref[...] + u_ref[...]
                  + di_ref[...] * bt_ref[...] + do_ref[...] * dt_ref[...])


_combine = pl.pallas_call(
    _comb_body,
    grid=(N_PAD // BN,),
    in_specs=[pl.BlockSpec((BN, D), lambda i: (i, 0)) for _ in range(5)]
    + [pl.BlockSpec((BN, 1), lambda i: (i, 0)) for _ in range(2)],
    out_specs=pl.BlockSpec((BN, D), lambda i: (i, 0)),
    out_shape=jax.ShapeDtypeStruct((N_PAD, D), jnp.float32),
)


def kernel(node_states, from_idx, to_idx, W_msg, b_msg, W_rmsg, b_rmsg, W_upd, b_upd):
    wcat, bcat = _prep(W_msg, W_rmsg, W_upd,
                       b_msg.reshape(1, D), b_rmsg.reshape(1, D), b_upd.reshape(1, D))
    ns_pad = jnp.pad(node_states, ((0, N_PAD - N), (0, 0)))
    a, c, bt, dt, u = _proj(ns_pad, wcat, bcat)
    idx_pad = N + (jnp.arange(E_PAD - E, dtype=jnp.int32) % (N_PAD - N))
    fi = jnp.concatenate([from_idx, idx_pad])
    ti = jnp.concatenate([to_idx, idx_pad])
    z2 = jnp.zeros((N_PAD, D), jnp.float32)
    z1 = jnp.zeros((N_PAD,), jnp.float32)
    s, degs = _sc_scatter(a, c, fi, ti, z2, z1)
    indeg = (degs[0, 0] + degs[1, 0]).reshape(N_PAD, 1)
    outdeg = (degs[0, 1] + degs[1, 1]).reshape(N_PAD, 1)
    return _combine(s[0], s[1], u, bt, dt, indeg, outdeg)[:N]
